# Initial kernel scaffold; baseline (speedup 1.0000x reference)
import jax
import jax.numpy as jnp
from jax.experimental import pallas as pl


def kernel(x_drug, x_protein, params, edge_index_dp, edge_index_pd, edge_label_index):
    raise NotImplementedError("write your pallas kernel here")


# speedup vs baseline: 10.7235x; 10.7235x over previous
import numpy as np
import jax
import jax.numpy as jnp
from jax.experimental import pallas as pl
from jax.experimental.pallas import tpu as pltpu

H = 2
D = 32
HID = 64
ND = 50000
NPR = 50000
NE = 600000
NL = 100000


def _bn(x, g, b):
    mu = jnp.mean(x, axis=0)
    var = jnp.var(x, axis=0)
    return g * (x - mu) / jnp.sqrt(var + 1e-5) + b


def _edge_phase(ka, q, vm, src, dst, num_dst):
    kae = ka[src]
    qe = q[dst]
    alpha = jnp.sum(kae.reshape(NE, H, D) * qe.reshape(NE, H, D), axis=-1)
    ea = jnp.exp(alpha)
    vme = vm[src].reshape(NE, H, D)
    contrib = (ea[..., None] * vme).reshape(NE, H * D)
    packed = jnp.concatenate([contrib, ea], axis=1)
    acc = jax.ops.segment_sum(packed, dst, num_segments=num_dst)
    num = acc[:, : H * D].reshape(num_dst, H, D)
    den = acc[:, H * D:]
    return (num / (den[..., None] + 1e-16)).reshape(num_dst, H * D)


def _mlp_block(e_ref, w1_ref, b1_ref, h_ref):
    h_ref[...] = jax.nn.relu(
        jnp.dot(e_ref[...], w1_ref[...], preferred_element_type=jnp.float32)
        + b1_ref[...]
    )


def kernel(x_drug, x_protein, params, edge_index_dp, edge_index_pd, edge_label_index):
    p = params
    hd = jax.nn.relu(_bn(x_drug @ p['lin_drug_w'].T + p['lin_drug_b'],
                         p['bn_drug_g'], p['bn_drug_b']))
    hp = jax.nn.relu(_bn(x_protein @ p['lin_prot_w'].T + p['lin_prot_b'],
                         p['bn_prot_g'], p['bn_prot_b']))

    for l in range(2):
        pre = 'l%d_' % l

        def bd(a):
            z = jnp.zeros((D, D), a.dtype)
            return jnp.block([[a[0], z], [z, a[1]]])

        scale_dp = jnp.repeat(p[pre + 'p_dp'], D) / np.sqrt(D)
        scale_pd = jnp.repeat(p[pre + 'p_pd'], D) / np.sqrt(D)
        a_dp = bd(p[pre + 'a_dp'])
        a_pd = bd(p[pre + 'a_pd'])
        m_dp = bd(p[pre + 'm_dp'])
        m_pd = bd(p[pre + 'm_pd'])

        ka_d = ((hd @ p[pre + 'k_drug_w'].T + p[pre + 'k_drug_b']) @ a_dp) * scale_dp
        vm_d = (hd @ p[pre + 'v_drug_w'].T + p[pre + 'v_drug_b']) @ m_dp
        q_p = hp @ p[pre + 'q_prot_w'].T + p[pre + 'q_prot_b']
        ka_p = ((hp @ p[pre + 'k_prot_w'].T + p[pre + 'k_prot_b']) @ a_pd) * scale_pd
        vm_p = (hp @ p[pre + 'v_prot_w'].T + p[pre + 'v_prot_b']) @ m_pd
        q_d = hd @ p[pre + 'q_drug_w'].T + p[pre + 'q_drug_b']

        agg_p = _edge_phase(ka_d, q_p, vm_d, edge_index_dp[0], edge_index_dp[1], NPR)
        agg_d = _edge_phase(ka_p, q_d, vm_p, edge_index_pd[0], edge_index_pd[1], ND)

        od = jax.nn.gelu(agg_d) @ p[pre + 'a_drug_w'].T + p[pre + 'a_drug_b']
        skd = jax.nn.sigmoid(p[pre + 'skip_drug'])
        hd = skd * od + (1.0 - skd) * hd
        op_ = jax.nn.gelu(agg_p) @ p[pre + 'a_prot_w'].T + p[pre + 'a_prot_b']
        skp = jax.nn.sigmoid(p[pre + 'skip_prot'])
        hp = skp * op_ + (1.0 - skp) * hp

    de = hd[edge_label_index[0]]
    pe = hp[edge_label_index[1]]
    e = jnp.concatenate([de, pe], axis=1)

    BLK = 2000
    h = pl.pallas_call(
        _mlp_block,
        grid=(NL // BLK,),
        in_specs=[
            pl.BlockSpec((BLK, 2 * HID), lambda i: (i, 0)),
            pl.BlockSpec((2 * HID, HID), lambda i: (0, 0)),
            pl.BlockSpec((HID,), lambda i: (0,)),
        ],
        out_specs=pl.BlockSpec((BLK, HID), lambda i: (i, 0)),
        out_shape=jax.ShapeDtypeStruct((NL, HID), jnp.float32),
    )(e, p['mlp_w1'].T, p['mlp_b1'])
    h = _bn(h, p['mlp_bn_g'], p['mlp_bn_b'])
    logits = h @ p['mlp_w2'].T + p['mlp_b2']
    return logits, hd, hp


# speedup vs baseline: 33.8431x; 3.1560x over previous
import functools

import numpy as np
import jax
import jax.numpy as jnp
from jax import lax
from jax.experimental import pallas as pl
from jax.experimental.pallas import tpu as pltpu
from jax.experimental.pallas import tpu_sc as plsc

H = 2
D = 32
HID = 64
ND = 50000
NPR = 50000
NE = 600000
NL = 100000

NC = 2
NS = 16

EPAD = 606208
TPT = EPAD // 32
TPS = EPAD // 16
ECH = 512
NCH_G = TPT // ECH
NCH_S = TPS // ECH
EROWS = EPAD // 128

HALF = 25000
RPT = 1564
HPAD = NS * RPT
DUMMY = 25008
CW = 40

NLPAD = 102400
LPT = NLPAD // 32
LCH = 640
LROWS = NLPAD // 128


def _mesh():
    return plsc.VectorSubcoreMesh(core_axis_name="c", subcore_axis_name="s",
                                  num_cores=NC, num_subcores=NS)


_SC_PARAMS = pltpu.CompilerParams(use_tc_tiling_on_sc=False)


def _sc_gather_body(sv_hbm, q_hbm, sidx_hbm, didx_hbm, gs_out, qd_out,
                    sidx_v, didx_v, gs_b, qd_b, sem1, sem2):
    c = lax.axis_index("c")
    s = lax.axis_index("s")
    wid = s * NC + c

    def chunk(g, carry):
        row0 = wid * (TPT // 128) + g * (ECH // 128)
        e0 = wid * TPT + g * ECH
        pltpu.sync_copy(sidx_hbm.at[pl.ds(row0, ECH // 128)], sidx_v)
        pltpu.sync_copy(didx_hbm.at[pl.ds(row0, ECH // 128)], didx_v)
        cps = []
        for j in range(ECH // 128):
            cps.append(pltpu.async_copy(
                sv_hbm.at[sidx_v.at[j]], gs_b.at[pl.ds(j * 128, 128)], sem1))
            cps.append(pltpu.async_copy(
                q_hbm.at[didx_v.at[j]], qd_b.at[pl.ds(j * 128, 128)], sem2))
        for cp in cps:
            cp.wait()
        pltpu.sync_copy(gs_b, gs_out.at[pl.ds(e0, ECH)])
        pltpu.sync_copy(qd_b, qd_out.at[pl.ds(e0, ECH)])
        return carry

    lax.fori_loop(0, NCH_G, chunk, 0)


def _sc_gather_edges(sv, qpad, sidx2, didx2):
    k = pl.kernel(
        _sc_gather_body,
        out_type=[jax.ShapeDtypeStruct((EPAD, 2 * HID), jnp.float32),
                  jax.ShapeDtypeStruct((EPAD, HID), jnp.float32)],
        mesh=_mesh(),
        scratch_types=[
            pltpu.VMEM((ECH // 128, 128), jnp.int32),
            pltpu.VMEM((ECH // 128, 128), jnp.int32),
            pltpu.VMEM((ECH, 2 * HID), jnp.float32),
            pltpu.VMEM((ECH, HID), jnp.float32),
            pltpu.SemaphoreType.DMA,
            pltpu.SemaphoreType.DMA,
        ],
        compiler_params=_SC_PARAMS,
    )
    return k(sv, qpad, sidx2, didx2)


def _sc_scatter_body(contrib_hbm, didx_hbm, zeros_hbm, acc_out,
                     didx_v, loc_v, cbuf, accS):
    c = lax.axis_index("c")
    s = lax.axis_index("s")
    coff = c * HALF
    pltpu.sync_copy(zeros_hbm.at[pl.ds(s * RPT, RPT)],
                    accS.at[pl.ds(s * RPT, RPT)])
    plsc.subcore_barrier()

    def chunk(g, carry):
        row0 = s * (TPS // 128) + g * (ECH // 128)
        e0 = s * TPS + g * ECH
        pltpu.sync_copy(didx_hbm.at[pl.ds(row0, ECH // 128)], didx_v)
        pltpu.sync_copy(contrib_hbm.at[pl.ds(e0, ECH)], cbuf)
        for j in range(ECH // 128):
            for t in range(8):
                v = didx_v[j, pl.ds(t * 16, 16)]
                loc = v - coff
                ok = (loc >= 0) & (loc < HALF)
                loc_v[j, pl.ds(t * 16, 16)] = jnp.where(ok, loc, DUMMY)
        for j in range(ECH // 128):
            pltpu.sync_copy(cbuf.at[pl.ds(j * 128, 128)],
                            accS.at[loc_v.at[j]], add=True)
        return carry

    lax.fori_loop(0, NCH_S, chunk, 0)
    plsc.subcore_barrier()
    pltpu.sync_copy(accS.at[pl.ds(s * RPT, RPT)],
                    acc_out.at[pl.ds(c * HPAD + s * RPT, RPT)])


def _sc_scatter_edges(contrib, didx2, zeros):
    k = pl.kernel(
        _sc_scatter_body,
        out_type=jax.ShapeDtypeStruct((2 * HPAD, CW), jnp.float32),
        mesh=_mesh(),
        scratch_types=[
            pltpu.VMEM((ECH // 128, 128), jnp.int32),
            pltpu.VMEM((ECH // 128, 128), jnp.int32),
            pltpu.VMEM((ECH, CW), jnp.float32),
            pltpu.VMEM_SHARED((HPAD, CW), jnp.float32),
        ],
        compiler_params=_SC_PARAMS,
    )
    return k(contrib, didx2, zeros)


def _sc_label_body(hd_hbm, hp_hbm, i0_hbm, i1_hbm, de_out, pe_out,
                   i0_v, i1_v, de_b, pe_b, sem1, sem2):
    c = lax.axis_index("c")
    s = lax.axis_index("s")
    wid = s * NC + c
    for g in range(LPT // LCH):
        row0 = wid * (LPT // 128) + g * (LCH // 128)
        e0 = wid * LPT + g * LCH
        pltpu.sync_copy(i0_hbm.at[pl.ds(row0, LCH // 128)], i0_v)
        pltpu.sync_copy(i1_hbm.at[pl.ds(row0, LCH // 128)], i1_v)
        cps = []
        for j in range(LCH // 128):
            cps.append(pltpu.async_copy(
                hd_hbm.at[i0_v.at[j]], de_b.at[pl.ds(j * 128, 128)], sem1))
            cps.append(pltpu.async_copy(
                hp_hbm.at[i1_v.at[j]], pe_b.at[pl.ds(j * 128, 128)], sem2))
        for cp in cps:
            cp.wait()
        pltpu.sync_copy(de_b, de_out.at[pl.ds(e0, LCH)])
        pltpu.sync_copy(pe_b, pe_out.at[pl.ds(e0, LCH)])


def _sc_label_gather(hd, hp, i0, i1):
    k = pl.kernel(
        _sc_label_body,
        out_type=[jax.ShapeDtypeStruct((NLPAD, HID), jnp.float32),
                  jax.ShapeDtypeStruct((NLPAD, HID), jnp.float32)],
        mesh=_mesh(),
        scratch_types=[
            pltpu.VMEM((LCH // 128, 128), jnp.int32),
            pltpu.VMEM((LCH // 128, 128), jnp.int32),
            pltpu.VMEM((LCH, HID), jnp.float32),
            pltpu.VMEM((LCH, HID), jnp.float32),
            pltpu.SemaphoreType.DMA,
            pltpu.SemaphoreType.DMA,
        ],
        compiler_params=_SC_PARAMS,
    )
    return k(hd, hp, i0, i1)


def _k_embed(x_ref, w_ref, b_ref, y_ref, st_ref):
    y = jnp.dot(x_ref[...], w_ref[...],
                preferred_element_type=jnp.float32) + b_ref[...]
    y_ref[...] = y

    @pl.when(pl.program_id(0) == 0)
    def _():
        st_ref[...] = jnp.zeros_like(st_ref)

    st_ref[0:1, :] += jnp.sum(y, axis=0, keepdims=True)
    st_ref[1:2, :] += jnp.sum(y * y, axis=0, keepdims=True)


def _embed(x, w, b, blk, n):
    return pl.pallas_call(
        _k_embed,
        grid=(n // blk,),
        in_specs=[
            pl.BlockSpec((blk, x.shape[1]), lambda i: (i, 0)),
            pl.BlockSpec(w.shape, lambda i: (0, 0)),
            pl.BlockSpec(b.shape, lambda i: (0,)),
        ],
        out_specs=[
            pl.BlockSpec((blk, HID), lambda i: (i, 0)),
            pl.BlockSpec((2, HID), lambda i: (0, 0)),
        ],
        out_shape=[jax.ShapeDtypeStruct((n, HID), jnp.float32),
                   jax.ShapeDtypeStruct((2, HID), jnp.float32)],
    )(x, w, b)


def _k_bnrelu(y_ref, st_ref, g_ref, b_ref, o_ref, *, n):
    s1 = st_ref[0:1, :]
    s2 = st_ref[1:2, :]
    mu = s1 / n
    var = s2 / n - mu * mu
    rstd = jax.lax.rsqrt(var + 1e-5)
    o_ref[...] = jax.nn.relu(g_ref[...] * (y_ref[...] - mu) * rstd + b_ref[...])


def _bnrelu(y, st, g, b, blk, n):
    return pl.pallas_call(
        functools.partial(_k_bnrelu, n=float(n)),
        grid=(n // blk,),
        in_specs=[
            pl.BlockSpec((blk, HID), lambda i: (i, 0)),
            pl.BlockSpec((2, HID), lambda i: (0, 0)),
            pl.BlockSpec((HID,), lambda i: (0,)),
            pl.BlockSpec((HID,), lambda i: (0,)),
        ],
        out_specs=pl.BlockSpec((blk, HID), lambda i: (i, 0)),
        out_shape=jax.ShapeDtypeStruct((n, HID), jnp.float32),
    )(y, st, g, b)


def _k_proj(h_ref, w_ref, b_ref, sv_ref, q_ref):
    y = jnp.dot(h_ref[...], w_ref[...],
                preferred_element_type=jnp.float32) + b_ref[...]
    sv_ref[...] = y[:, :2 * HID]
    q_ref[...] = y[:, 2 * HID:]


def _proj(h, w, b, blk, n):
    return pl.pallas_call(
        _k_proj,
        grid=(n // blk,),
        in_specs=[
            pl.BlockSpec((blk, HID), lambda i: (i, 0)),
            pl.BlockSpec((HID, 3 * HID), lambda i: (0, 0)),
            pl.BlockSpec((3 * HID,), lambda i: (0,)),
        ],
        out_specs=[
            pl.BlockSpec((blk, 2 * HID), lambda i: (i, 0)),
            pl.BlockSpec((blk, HID), lambda i: (i, 0)),
        ],
        out_shape=[jax.ShapeDtypeStruct((n, 2 * HID), jnp.float32),
                   jax.ShapeDtypeStruct((n, HID), jnp.float32)],
    )(h, w, b)


def _k_edge(gs_ref, qd_ref, c0_ref, c1_ref):
    gs = gs_ref[...]
    qd = qd_ref[...]
    for h, ref in ((0, c0_ref), (1, c1_ref)):
        ka = gs[:, h * D:(h + 1) * D]
        vm = gs[:, 2 * D + h * D:2 * D + (h + 1) * D]
        qh = qd[:, h * D:(h + 1) * D]
        alpha = jnp.sum(ka * qh, axis=1, keepdims=True)
        ea = jnp.exp(alpha)
        pad = jnp.zeros((ka.shape[0], CW - D - 1), jnp.float32)
        ref[...] = jnp.concatenate([ea * vm, ea, pad], axis=1)


def _edge_tc(gs, qd, blk=1024):
    return pl.pallas_call(
        _k_edge,
        grid=(EPAD // blk,),
        in_specs=[
            pl.BlockSpec((blk, 2 * HID), lambda i: (i, 0)),
            pl.BlockSpec((blk, HID), lambda i: (i, 0)),
        ],
        out_specs=[pl.BlockSpec((blk, CW), lambda i: (i, 0)),
                   pl.BlockSpec((blk, CW), lambda i: (i, 0))],
        out_shape=[jax.ShapeDtypeStruct((EPAD, CW), jnp.float32),
                   jax.ShapeDtypeStruct((EPAD, CW), jnp.float32)],
    )(gs, qd)


def _k_post(a0_ref, a1_ref, h_ref, w_ref, b_ref, sk_ref, o_ref):
    a0 = a0_ref[...]
    a1 = a1_ref[...]
    num = jnp.concatenate([a0[:, :D], a1[:, :D]], axis=1)
    dd = jnp.concatenate(
        [jnp.broadcast_to(a[:, D:D + 1], a[:, :D].shape) for a in (a0, a1)],
        axis=1)
    agg = num / (dd + 1e-16)
    od = jnp.dot(jax.nn.gelu(agg), w_ref[...],
                 preferred_element_type=jnp.float32) + b_ref[...]
    sk = sk_ref[...]
    o_ref[...] = sk * od + (1.0 - sk) * h_ref[...]


def _post(acc, h, w, b, skv, blk, n):
    return pl.pallas_call(
        _k_post,
        grid=(n // blk,),
        in_specs=[
            pl.BlockSpec((blk, CW), lambda i: (i, 0)),
            pl.BlockSpec((blk, CW), lambda i: (i, 0)),
            pl.BlockSpec((blk, HID), lambda i: (i, 0)),
            pl.BlockSpec((HID, HID), lambda i: (0, 0)),
            pl.BlockSpec((HID,), lambda i: (0,)),
            pl.BlockSpec((HID,), lambda i: (0,)),
        ],
        out_specs=pl.BlockSpec((blk, HID), lambda i: (i, 0)),
        out_shape=jax.ShapeDtypeStruct((n, HID), jnp.float32),
    )(acc[0], acc[1], h, w, b, skv)


def _k_mlp1(de_ref, pe_ref, wa_ref, wb_ref, b_ref, h_ref, st_ref):
    y = (jnp.dot(de_ref[...], wa_ref[...], preferred_element_type=jnp.float32)
         + jnp.dot(pe_ref[...], wb_ref[...], preferred_element_type=jnp.float32)
         + b_ref[...])
    y = jax.nn.relu(y)
    h_ref[...] = y

    @pl.when(pl.program_id(0) == 0)
    def _():
        st_ref[...] = jnp.zeros_like(st_ref)

    st_ref[0:1, :] += jnp.sum(y, axis=0, keepdims=True)
    st_ref[1:2, :] += jnp.sum(y * y, axis=0, keepdims=True)


def _mlp1(de, pe, wa, wb, b, blk):
    return pl.pallas_call(
        _k_mlp1,
        grid=(NL // blk,),
        in_specs=[
            pl.BlockSpec((blk, HID), lambda i: (i, 0)),
            pl.BlockSpec((blk, HID), lambda i: (i, 0)),
            pl.BlockSpec((HID, HID), lambda i: (0, 0)),
            pl.BlockSpec((HID, HID), lambda i: (0, 0)),
            pl.BlockSpec((HID,), lambda i: (0,)),
        ],
        out_specs=[
            pl.BlockSpec((blk, HID), lambda i: (i, 0)),
            pl.BlockSpec((2, HID), lambda i: (0, 0)),
        ],
        out_shape=[jax.ShapeDtypeStruct((NL, HID), jnp.float32),
                   jax.ShapeDtypeStruct((2, HID), jnp.float32)],
    )(de, pe, wa, wb, b)


def _k_mlp2(h_ref, st_ref, g_ref, b_ref, w2_ref, b2_ref, o_ref, *, n):
    s1 = st_ref[0:1, :]
    s2 = st_ref[1:2, :]
    mu = s1 / n
    var = s2 / n - mu * mu
    rstd = jax.lax.rsqrt(var + 1e-5)
    hn = g_ref[...] * (h_ref[...] - mu) * rstd + b_ref[...]
    o_ref[...] = jnp.dot(hn, w2_ref[...],
                         preferred_element_type=jnp.float32) + b2_ref[...]


def _mlp2(h, st, g, b, w2, b2, blk):
    return pl.pallas_call(
        functools.partial(_k_mlp2, n=float(NL)),
        grid=(NL // blk,),
        in_specs=[
            pl.BlockSpec((blk, HID), lambda i: (i, 0)),
            pl.BlockSpec((2, HID), lambda i: (0, 0)),
            pl.BlockSpec((HID,), lambda i: (0,)),
            pl.BlockSpec((HID,), lambda i: (0,)),
            pl.BlockSpec((HID, 1), lambda i: (0, 0)),
            pl.BlockSpec((1,), lambda i: (0,)),
        ],
        out_specs=pl.BlockSpec((blk, 1), lambda i: (i, 0)),
        out_shape=jax.ShapeDtypeStruct((NL, 1), jnp.float32),
    )(h, st, g, b, w2, b2)


def _pad_idx(a, padval, total):
    pad = jnp.full((total - a.shape[0],), padval, a.dtype)
    return jnp.concatenate([a, pad]).reshape(-1, 128)


def _edge_phase(sv, q, sidx2, didx2, zeros):
    qpad = jnp.concatenate([q, jnp.zeros((8, HID), jnp.float32)])
    gs, qd = _sc_gather_edges(sv, qpad, sidx2, didx2)
    c0, c1 = _edge_tc(gs, qd)
    accs = []
    for ch in (c0, c1):
        accH = _sc_scatter_edges(ch, didx2, zeros)
        accs.append(jnp.concatenate([accH[0:HALF], accH[HPAD:HPAD + HALF]]))
    return accs


def kernel(x_drug, x_protein, params, edge_index_dp, edge_index_pd, edge_label_index):
    p = params
    f32 = jnp.float32

    sidx_dp = _pad_idx(edge_index_dp[0].astype(jnp.int32), 0, EPAD)
    didx_dp = _pad_idx(edge_index_dp[1].astype(jnp.int32), NPR, EPAD)
    sidx_pd = _pad_idx(edge_index_pd[0].astype(jnp.int32), 0, EPAD)
    didx_pd = _pad_idx(edge_index_pd[1].astype(jnp.int32), ND, EPAD)
    li0 = _pad_idx(edge_label_index[0].astype(jnp.int32), 0, NLPAD)
    li1 = _pad_idx(edge_label_index[1].astype(jnp.int32), 0, NLPAD)
    zeros = jnp.zeros((HPAD, CW), f32)

    yd, std = _embed(x_drug, p['lin_drug_w'].T, p['lin_drug_b'], 1000, ND)
    hd = _bnrelu(yd, std, p['bn_drug_g'], p['bn_drug_b'], 1000, ND)
    yp, stp = _embed(x_protein, p['lin_prot_w'].T, p['lin_prot_b'], 1000, NPR)
    hp = _bnrelu(yp, stp, p['bn_prot_g'], p['bn_prot_b'], 1000, NPR)

    for l in range(2):
        pre = 'l%d_' % l

        def bd(a):
            z = jnp.zeros((D, D), f32)
            return jnp.block([[a[0], z], [z, a[1]]])

        scale_dp = jnp.repeat(p[pre + 'p_dp'], D) / np.sqrt(D)
        scale_pd = jnp.repeat(p[pre + 'p_pd'], D) / np.sqrt(D)

        def fuse(kw, kb, vw, vb, qw, qb, a_bd, m_bd, scale):
            wka = (kw.T @ a_bd) * scale
            bka = (kb @ a_bd) * scale
            wvm = vw.T @ m_bd
            bvm = vb @ m_bd
            w = jnp.concatenate([wka, wvm, qw.T], axis=1)
            b = jnp.concatenate([bka, bvm, qb])
            return w, b

        wd, bdg = fuse(p[pre + 'k_drug_w'], p[pre + 'k_drug_b'],
                       p[pre + 'v_drug_w'], p[pre + 'v_drug_b'],
                       p[pre + 'q_drug_w'], p[pre + 'q_drug_b'],
                       bd(p[pre + 'a_dp']), bd(p[pre + 'm_dp']), scale_dp)
        wp, bpg = fuse(p[pre + 'k_prot_w'], p[pre + 'k_prot_b'],
                       p[pre + 'v_prot_w'], p[pre + 'v_prot_b'],
                       p[pre + 'q_prot_w'], p[pre + 'q_prot_b'],
                       bd(p[pre + 'a_pd']), bd(p[pre + 'm_pd']), scale_pd)

        sv_d, q_d = _proj(hd, wd, bdg, 1000, ND)
        sv_p, q_p = _proj(hp, wp, bpg, 1000, NPR)

        acc_p = _edge_phase(sv_d, q_p, sidx_dp, didx_dp, zeros)
        acc_d = _edge_phase(sv_p, q_d, sidx_pd, didx_pd, zeros)

        skd = jnp.broadcast_to(jax.nn.sigmoid(p[pre + 'skip_drug']), (HID,))
        skp = jnp.broadcast_to(jax.nn.sigmoid(p[pre + 'skip_prot']), (HID,))
        hd = _post(acc_d, hd, p[pre + 'a_drug_w'].T, p[pre + 'a_drug_b'],
                   skd, 1000, ND)
        hp = _post(acc_p, hp, p[pre + 'a_prot_w'].T, p[pre + 'a_prot_b'],
                   skp, 1000, NPR)

    de, pe = _sc_label_gather(hd, hp, li0, li1)
    w1t = p['mlp_w1'].T
    h, sth = _mlp1(de, pe, w1t[:HID], w1t[HID:], p['mlp_b1'], 1000)
    logits = _mlp2(h, sth, p['mlp_bn_g'], p['mlp_bn_b'],
                   p['mlp_w2'].T, p['mlp_b2'], 1000)
    return logits, hd, hp
